# baseline (speedup 1.0000x reference)
import functools

import jax
import jax.numpy as jnp
from jax import lax
from jax.experimental import pallas as pl
from jax.experimental.pallas import tpu as pltpu
from jax.experimental.pallas import tpu_sc as plsc

_E = 8
_K = 2
_H = 1024
_I = 1024
_R = 16
_T = 2048
_BT = 256
_NB = (_T * _K) // _BT + _E
_P = _NB * _BT
_NW = 32
_SPT = _P // _NW
_TPT = _T // _NW
_GCH = 48
_CCH = 32

_MESH = dict(core_axis_name="c", subcore_axis_name="s", num_cores=2,
             num_subcores=16)


def _meta_body(ids_ref, tw_ref, dst_ref, be_ref, stok_ref, sw_ref):
    f32 = jnp.float32
    iota_e = lax.broadcasted_iota(jnp.int32, (1, _E), 1)
    rows = lax.broadcasted_iota(jnp.int32, (_BT, _BT), 0)
    cols = lax.broadcasted_iota(jnp.int32, (_BT, _BT), 1)
    tri = (rows > cols).astype(f32)
    dn = (((1,), (0,)), ((), ()))

    def scan_ranks(col):
        carry = jnp.zeros((1, _E), f32)
        oh_tiles, excl_tiles = [], []
        for i in range(_T // _BT):
            ids_i = ids_ref[pl.ds(i * _BT, _BT), col:col + 1]
            oh = (ids_i == iota_e).astype(f32)
            excl = lax.dot_general(tri, oh, dn,
                                   preferred_element_type=f32) + carry
            carry = carry + jnp.sum(oh, axis=0, keepdims=True)
            oh_tiles.append(oh)
            excl_tiles.append(excl)
        return (jnp.concatenate(oh_tiles, axis=0),
                jnp.concatenate(excl_tiles, axis=0), carry)

    oh0, excl0, tot0 = scan_ranks(0)
    oh1, excl1, tot1 = scan_ranks(1)
    counts = tot0 + tot1
    nblk = jnp.floor((counts + float(_BT - 1)) / float(_BT))
    lt = (lax.broadcasted_iota(jnp.int32, (_E, _E), 0) <
          lax.broadcasted_iota(jnp.int32, (_E, _E), 1)).astype(f32)
    blk_off = lax.dot_general(nblk, lt, dn, preferred_element_type=f32)
    pad_off = blk_off * float(_BT)

    dst0 = jnp.sum(oh0 * (pad_off + excl0), axis=1, keepdims=True)
    dst1 = jnp.sum(oh1 * (pad_off + tot0 + excl1), axis=1, keepdims=True)
    dst_ref[...] = jnp.concatenate([dst0, dst1], axis=1).astype(jnp.int32)

    tw = tw_ref[...]
    tww = tw / jnp.sum(tw, axis=1, keepdims=True)

    dst_all = jnp.concatenate([dst0, dst1], axis=0)
    toks_f = lax.broadcasted_iota(jnp.int32, (_T, 1), 0).astype(f32)
    toks2 = jnp.concatenate([toks_f, toks_f], axis=0)
    pw2 = jnp.concatenate([tww[:, 0:1], tww[:, 1:2]], axis=0)
    tv = jnp.concatenate([toks2, pw2], axis=1)
    slot_iota = lax.broadcasted_iota(jnp.int32, (1, _BT), 1).astype(f32)
    dnp = (((0,), (0,)), ((), ()))
    for b in range(_NB):
        ind = (dst_all == (float(b * _BT) + slot_iota)).astype(f32)
        res = lax.dot_general(ind, tv, dnp, preferred_element_type=f32,
                              precision=lax.Precision.HIGHEST)
        stok_ref[pl.ds(b * _BT, _BT), :] = res[:, 0:1].astype(jnp.int32)
        sw_ref[pl.ds(b * _BT, _BT), :] = res[:, 1:2]

    bi = lax.broadcasted_iota(jnp.int32, (_NB, _E), 0).astype(f32)
    ge = (bi >= jnp.broadcast_to(blk_off, (_NB, _E))).astype(f32)
    be_val = jnp.sum(ge, axis=1, keepdims=True) - 1.0
    total_blk = jnp.sum(nblk, axis=1, keepdims=True)
    active = bi[:, 0:1] < jnp.broadcast_to(total_blk, (_NB, 1))
    be_ref[...] = jnp.where(active, be_val, -1.0).astype(jnp.int32)


def _run_meta(topk_ids, topk_weights):
    return pl.pallas_call(
        _meta_body,
        out_shape=(
            jax.ShapeDtypeStruct((_T, _K), jnp.int32),
            jax.ShapeDtypeStruct((_NB, 1), jnp.int32),
            jax.ShapeDtypeStruct((_P, 1), jnp.int32),
            jax.ShapeDtypeStruct((_P, 1), jnp.float32),
        ),
    )(topk_ids, topk_weights)


@functools.lru_cache(maxsize=None)
def _make_dispatch():
    return functools.partial(
        pl.kernel,
        out_type=jax.ShapeDtypeStruct((_P, _H), jnp.float32),
        mesh=plsc.VectorSubcoreMesh(**_MESH),
        compiler_params=pltpu.CompilerParams(needs_layout_passes=False),
        scratch_types=[
            pltpu.VMEM((_SPT,), jnp.int32),
            pltpu.VMEM((_GCH, _H), jnp.float32),
            pltpu.VMEM((_GCH, _H), jnp.float32),
            pltpu.SemaphoreType.DMA,
            pltpu.SemaphoreType.DMA,
        ],
    )(_dispatch_body)


def _dispatch_body(stok_hbm, hid_hbm, xs_hbm, tok_v, rows0, rows1,
                   sem_g, sem_w):
    cid = lax.axis_index("c")
    sid = lax.axis_index("s")
    wid = sid * 2 + cid
    base = wid * _SPT
    pltpu.sync_copy(stok_hbm.at[pl.ds(base, _SPT)], tok_v)

    with jax.named_scope("disp_gather"):
        nch = _SPT // _GCH
        bufs = [rows0, rows1]
        gathers = [None] * nch
        writes = [None] * nch
        gathers[0] = pltpu.async_copy(hid_hbm.at[tok_v.at[pl.ds(0, _GCH)]],
                                      bufs[0], sem_g)
        if nch > 1:
            gathers[1] = pltpu.async_copy(
                hid_hbm.at[tok_v.at[pl.ds(_GCH, _GCH)]], bufs[1], sem_g)
        for i in range(nch):
            gathers[i].wait()
            writes[i] = pltpu.async_copy(
                bufs[i % 2], xs_hbm.at[pl.ds(base + i * _GCH, _GCH)], sem_w)
            if i + 2 < nch:
                writes[i].wait()
                gathers[i + 2] = pltpu.async_copy(
                    hid_hbm.at[tok_v.at[pl.ds((i + 2) * _GCH, _GCH)]],
                    bufs[i % 2], sem_g)
        for i in range(max(0, nch - 2), nch):
            writes[i].wait()


def _conv_body(w13_ref, w2_ref, gua_ref, gub_ref, da_ref, db_ref,
               ow13_ref, ow2_ref, ogua_ref, ogub_ref, oda_ref, odb_ref):
    bf = jnp.bfloat16
    ow13_ref[...] = w13_ref[...].astype(bf)
    ow2_ref[...] = w2_ref[...].astype(bf)
    ogua_ref[...] = gua_ref[...].astype(bf)
    ogub_ref[...] = gub_ref[...].astype(bf)
    oda_ref[...] = da_ref[...].astype(bf)
    odb_ref[...] = db_ref[...].astype(bf)


def _run_convert(w13, w2, gua, gub, da, db):
    emap = lambda e: (e, 0, 0)
    specs = [
        pl.BlockSpec((1, 2 * _I, _H), emap),
        pl.BlockSpec((1, _H, _I), emap),
        pl.BlockSpec((1, _R, _H), emap),
        pl.BlockSpec((1, 2 * _I, _R), emap),
        pl.BlockSpec((1, _R, _I), emap),
        pl.BlockSpec((1, _H, _R), emap),
    ]
    bf = jnp.bfloat16
    return pl.pallas_call(
        _conv_body,
        grid=(_E,),
        in_specs=specs,
        out_specs=specs,
        out_shape=(
            jax.ShapeDtypeStruct((_E, 2 * _I, _H), bf),
            jax.ShapeDtypeStruct((_E, _H, _I), bf),
            jax.ShapeDtypeStruct((_E, _R, _H), bf),
            jax.ShapeDtypeStruct((_E, 2 * _I, _R), bf),
            jax.ShapeDtypeStruct((_E, _R, _I), bf),
            jax.ShapeDtypeStruct((_E, _H, _R), bf),
        ),
    )(w13, w2, gua, gub, da, db)


def _gmm_body(be_ref, xs_ref, sw_ref, w13_ref, w2_ref, gua_ref, gub_ref,
              da_ref, db_ref, out_ref):
    i = pl.program_id(0)
    active = be_ref[i] >= 0

    @pl.when(active)
    def _():
        bf = jnp.bfloat16
        x = xs_ref[...].astype(bf)
        dn = (((1,), (1,)), ((), ()))
        gate_up = lax.dot_general(x, w13_ref[0], dn,
                                  preferred_element_type=jnp.float32)
        mid = lax.dot_general(x, gua_ref[0], dn,
                              preferred_element_type=jnp.float32)
        gate_up = gate_up + lax.dot_general(mid.astype(bf), gub_ref[0], dn,
                                            preferred_element_type=jnp.float32)
        gate = gate_up[:, :_I]
        up = gate_up[:, _I:]
        act = (gate * jax.nn.sigmoid(gate) * up).astype(bf)
        down = lax.dot_general(act, w2_ref[0], dn,
                               preferred_element_type=jnp.float32)
        dmid = lax.dot_general(act, da_ref[0], dn,
                               preferred_element_type=jnp.float32)
        down = down + lax.dot_general(dmid.astype(bf), db_ref[0], dn,
                                      preferred_element_type=jnp.float32)
        out_ref[...] = sw_ref[...] * down

    @pl.when(jnp.logical_not(active))
    def _():
        out_ref[...] = jnp.zeros_like(out_ref)


def _run_gmm(be, xs, sw2d, w13, w2, gua, gub, da, db):
    def emap(i, be_r):
        return (jnp.where(be_r[i] < 0, _E - 1, be_r[i]), 0, 0)

    grid_spec = pltpu.PrefetchScalarGridSpec(
        num_scalar_prefetch=1,
        grid=(_NB,),
        in_specs=[
            pl.BlockSpec((_BT, _H), lambda i, be_r: (i, 0)),
            pl.BlockSpec((_BT, 1), lambda i, be_r: (i, 0)),
            pl.BlockSpec((1, 2 * _I, _H), emap),
            pl.BlockSpec((1, _H, _I), emap),
            pl.BlockSpec((1, _R, _H), emap),
            pl.BlockSpec((1, 2 * _I, _R), emap),
            pl.BlockSpec((1, _R, _I), emap),
            pl.BlockSpec((1, _H, _R), emap),
        ],
        out_specs=pl.BlockSpec((_BT, _H), lambda i, be_r: (i, 0)),
    )
    return pl.pallas_call(
        _gmm_body,
        grid_spec=grid_spec,
        out_shape=jax.ShapeDtypeStruct((_P, _H), jnp.float32),
    )(be, xs, sw2d, w13, w2, gua, gub, da, db)


@functools.lru_cache(maxsize=None)
def _make_combine():
    return functools.partial(
        pl.kernel,
        out_type=jax.ShapeDtypeStruct((_T, _H), jnp.float32),
        mesh=plsc.VectorSubcoreMesh(**_MESH),
        compiler_params=pltpu.CompilerParams(needs_layout_passes=False),
        scratch_types=[
            pltpu.VMEM((_TPT,), jnp.int32),
            pltpu.VMEM((_TPT,), jnp.int32),
            pltpu.VMEM((_CCH, _H), jnp.float32),
            pltpu.VMEM((_CCH, _H), jnp.float32),
            pltpu.SemaphoreType.DMA,
            pltpu.SemaphoreType.DMA,
        ],
    )(_combine_body)


def _combine_body(dstT_hbm, ys_hbm, out_hbm, idxA_v, idxB_v, bufA, bufB,
                  semA, semB):
    wid = lax.axis_index("s") * 2 + lax.axis_index("c")
    tbase = wid * _TPT
    pltpu.sync_copy(dstT_hbm.at[0, pl.ds(tbase, _TPT)], idxA_v)
    pltpu.sync_copy(dstT_hbm.at[1, pl.ds(tbase, _TPT)], idxB_v)

    for c2 in range(_TPT // _CCH):
        cpA = pltpu.async_copy(ys_hbm.at[idxA_v.at[pl.ds(c2 * _CCH, _CCH)]],
                               bufA, semA)
        cpB = pltpu.async_copy(ys_hbm.at[idxB_v.at[pl.ds(c2 * _CCH, _CCH)]],
                               bufB, semB)
        cpA.wait()
        cpB.wait()

        def add_body(j, _):
            r = j >> 6
            c = j & 63
            plsc.addupdate(bufA.at[r, pl.ds(c * 16, 16)],
                           bufB[r, pl.ds(c * 16, 16)])
            return 0
        lax.fori_loop(0, _CCH * (_H // 16), add_body, 0)
        pltpu.sync_copy(bufA, out_hbm.at[pl.ds(tbase + c2 * _CCH, _CCH)])


@jax.jit
def kernel(hidden_states, topk_weights, topk_ids, w13, w2, gate_up_lora_a,
           gate_up_lora_b, down_lora_a, down_lora_b):
    dst, be, stok, sw2d = _run_meta(topk_ids, topk_weights)
    dst_t = dst.T
    be24 = be.reshape(-1)
    xs = _make_dispatch()(stok.reshape(_P), hidden_states)
    wb = _run_convert(w13, w2, gate_up_lora_a, gate_up_lora_b,
                      down_lora_a, down_lora_b)
    ys = _run_gmm(be24, xs, sw2d, *wb)
    return _make_combine()(dst_t, ys)


# speedup vs baseline: 1.3832x; 1.3832x over previous
import functools

import jax
import jax.numpy as jnp
from jax import lax
from jax.experimental import pallas as pl
from jax.experimental.pallas import tpu as pltpu
from jax.experimental.pallas import tpu_sc as plsc

_E = 8
_K = 2
_H = 1024
_I = 1024
_R = 16
_T = 2048
_BT = 256
_NB = (_T * _K) // _BT + _E
_P = _NB * _BT
_NW = 32
_SPT = _P // _NW
_TPT = _T // _NW
_GCH = 48
_CCH = 32

_MESH = dict(core_axis_name="c", subcore_axis_name="s", num_cores=2,
             num_subcores=16)


def _meta_body(idst_ref, tw_ref, dstt_ref, be_ref, stok_ref, sw_ref):
    f32 = jnp.float32
    i32 = jnp.int32
    dn = (((1,), (0,)), ((), ()))

    iota_e_col = lax.broadcasted_iota(i32, (_E, 1), 0)
    triu = (lax.broadcasted_iota(i32, (_BT, _BT), 0) <
            lax.broadcasted_iota(i32, (_BT, _BT), 1)).astype(f32)
    ohs, excls, tots = [], [], []
    for r in range(_K):
        oh = (idst_ref[r:r + 1, :] == iota_e_col).astype(f32)
        carry = jnp.zeros((_E, 1), f32)
        tiles = []
        for i in range(_T // _BT):
            oht = oh[:, i * _BT:(i + 1) * _BT]
            tiles.append(lax.dot_general(oht, triu, dn,
                                         preferred_element_type=f32) + carry)
            carry = carry + jnp.sum(oht, axis=1, keepdims=True)
        ohs.append(oh)
        excls.append(jnp.concatenate(tiles, axis=1))
        tots.append(carry)

    counts = tots[0] + tots[1]
    nblk = jnp.floor((counts + float(_BT - 1)) / float(_BT))
    ltt = (lax.broadcasted_iota(i32, (_E, _E), 1) <
           lax.broadcasted_iota(i32, (_E, _E), 0)).astype(f32)
    blk_off = lax.dot_general(ltt, nblk, dn, preferred_element_type=f32)
    pad_off = blk_off * float(_BT)

    dst0 = jnp.sum(ohs[0] * (pad_off + excls[0]), axis=0, keepdims=True)
    dst1 = jnp.sum(ohs[1] * (pad_off + tots[0] + excls[1]), axis=0,
                   keepdims=True)
    dstt_ref[...] = jnp.concatenate([dst0, dst1], axis=0).astype(i32)

    tw = tw_ref[...]
    tww = tw / jnp.sum(tw, axis=1, keepdims=True)
    pw2 = jnp.concatenate([tww[:, 0:1], tww[:, 1:2]], axis=0)
    whi = pw2.astype(jnp.bfloat16).astype(f32)
    wlo = pw2 - whi
    tok = lax.broadcasted_iota(i32, (_T * _K, 1), 0) & (_T - 1)
    thi = (tok >> 7).astype(f32)
    tlo = (tok & 127).astype(f32)
    tv = jnp.concatenate([thi, tlo, whi, wlo], axis=1)

    dst_all = jnp.concatenate([dst0, dst1], axis=1)
    slot_col = lax.broadcasted_iota(i32, (_BT, 1), 0).astype(f32)
    for b in range(_NB):
        ind = (dst_all == (float(b * _BT) + slot_col)).astype(f32)
        res = lax.dot_general(ind, tv, dn, preferred_element_type=f32)
        stok_ref[pl.ds(b * _BT, _BT), :] = (
            res[:, 0:1] * 128.0 + res[:, 1:2]).astype(i32)
        sw_ref[pl.ds(b * _BT, _BT), :] = res[:, 2:3] + res[:, 3:4]

    bi = lax.broadcasted_iota(i32, (_E, _NB), 1).astype(f32)
    ge = (bi >= jnp.broadcast_to(blk_off, (_E, _NB))).astype(f32)
    be_val = jnp.sum(ge, axis=0, keepdims=True) - 1.0
    total_blk = jnp.sum(nblk, axis=0, keepdims=True)
    active = bi[0:1, :] < jnp.broadcast_to(total_blk, (1, _NB))
    be_ref[...] = jnp.where(active, be_val, -1.0).astype(i32)


def _run_meta(topk_ids_t, topk_weights):
    return pl.pallas_call(
        _meta_body,
        out_shape=(
            jax.ShapeDtypeStruct((_K, _T), jnp.int32),
            jax.ShapeDtypeStruct((1, _NB), jnp.int32),
            jax.ShapeDtypeStruct((_P, 1), jnp.int32),
            jax.ShapeDtypeStruct((_P, 1), jnp.float32),
        ),
    )(topk_ids_t, topk_weights)


@functools.lru_cache(maxsize=None)
def _make_dispatch():
    return functools.partial(
        pl.kernel,
        out_type=jax.ShapeDtypeStruct((_P, _H), jnp.float32),
        mesh=plsc.VectorSubcoreMesh(**_MESH),
        compiler_params=pltpu.CompilerParams(needs_layout_passes=False),
        scratch_types=[
            pltpu.VMEM((_SPT,), jnp.int32),
            pltpu.VMEM((_GCH, _H), jnp.float32),
            pltpu.VMEM((_GCH, _H), jnp.float32),
            pltpu.SemaphoreType.DMA,
            pltpu.SemaphoreType.DMA,
        ],
    )(_dispatch_body)


def _dispatch_body(stok_hbm, hid_hbm, xs_hbm, tok_v, rows0, rows1,
                   sem_g, sem_w):
    cid = lax.axis_index("c")
    sid = lax.axis_index("s")
    wid = sid * 2 + cid
    base = wid * _SPT
    pltpu.sync_copy(stok_hbm.at[pl.ds(base, _SPT)], tok_v)

    with jax.named_scope("disp_gather"):
        nch = _SPT // _GCH
        bufs = [rows0, rows1]
        gathers = [None] * nch
        writes = [None] * nch
        gathers[0] = pltpu.async_copy(hid_hbm.at[tok_v.at[pl.ds(0, _GCH)]],
                                      bufs[0], sem_g)
        if nch > 1:
            gathers[1] = pltpu.async_copy(
                hid_hbm.at[tok_v.at[pl.ds(_GCH, _GCH)]], bufs[1], sem_g)
        for i in range(nch):
            gathers[i].wait()
            writes[i] = pltpu.async_copy(
                bufs[i % 2], xs_hbm.at[pl.ds(base + i * _GCH, _GCH)], sem_w)
            if i + 2 < nch:
                writes[i].wait()
                gathers[i + 2] = pltpu.async_copy(
                    hid_hbm.at[tok_v.at[pl.ds((i + 2) * _GCH, _GCH)]],
                    bufs[i % 2], sem_g)
        for i in range(max(0, nch - 2), nch):
            writes[i].wait()


def _gmm_body(be_ref, xs_ref, sw_ref, w13_ref, w2_ref, gua_ref, gub_ref,
              da_ref, db_ref, out_ref):
    i = pl.program_id(0)
    active = be_ref[i] >= 0

    @pl.when(active)
    def _():
        bf = jnp.bfloat16
        x = xs_ref[...].astype(bf)
        dn = (((1,), (1,)), ((), ()))
        gate_up = lax.dot_general(x, w13_ref[0].astype(bf), dn,
                                  preferred_element_type=jnp.float32)
        mid = lax.dot_general(x, gua_ref[0].astype(bf), dn,
                              preferred_element_type=jnp.float32)
        gate_up = gate_up + lax.dot_general(mid.astype(bf),
                                            gub_ref[0].astype(bf), dn,
                                            preferred_element_type=jnp.float32)
        gate = gate_up[:, :_I]
        up = gate_up[:, _I:]
        act = (gate * jax.nn.sigmoid(gate) * up).astype(bf)
        down = lax.dot_general(act, w2_ref[0].astype(bf), dn,
                               preferred_element_type=jnp.float32)
        dmid = lax.dot_general(act, da_ref[0].astype(bf), dn,
                               preferred_element_type=jnp.float32)
        down = down + lax.dot_general(dmid.astype(bf), db_ref[0].astype(bf),
                                      dn, preferred_element_type=jnp.float32)
        out_ref[...] = sw_ref[...] * down

    @pl.when(jnp.logical_not(active))
    def _():
        out_ref[...] = jnp.zeros_like(out_ref)


def _run_gmm(be, xs, sw2d, w13, w2, gua, gub, da, db):
    def emap(i, be_r):
        return (jnp.where(be_r[i] < 0, _E - 1, be_r[i]), 0, 0)

    grid_spec = pltpu.PrefetchScalarGridSpec(
        num_scalar_prefetch=1,
        grid=(_NB,),
        in_specs=[
            pl.BlockSpec((_BT, _H), lambda i, be_r: (i, 0)),
            pl.BlockSpec((_BT, 1), lambda i, be_r: (i, 0)),
            pl.BlockSpec((1, 2 * _I, _H), emap),
            pl.BlockSpec((1, _H, _I), emap),
            pl.BlockSpec((1, _R, _H), emap),
            pl.BlockSpec((1, 2 * _I, _R), emap),
            pl.BlockSpec((1, _R, _I), emap),
            pl.BlockSpec((1, _H, _R), emap),
        ],
        out_specs=pl.BlockSpec((_BT, _H), lambda i, be_r: (i, 0)),
    )
    return pl.pallas_call(
        _gmm_body,
        grid_spec=grid_spec,
        out_shape=jax.ShapeDtypeStruct((_P, _H), jnp.float32),
    )(be, xs, sw2d, w13, w2, gua, gub, da, db)


@functools.lru_cache(maxsize=None)
def _make_combine():
    return functools.partial(
        pl.kernel,
        out_type=jax.ShapeDtypeStruct((_T, _H), jnp.float32),
        mesh=plsc.VectorSubcoreMesh(**_MESH),
        compiler_params=pltpu.CompilerParams(needs_layout_passes=False),
        scratch_types=[
            pltpu.VMEM((_TPT,), jnp.int32),
            pltpu.VMEM((_TPT,), jnp.int32),
            pltpu.VMEM((_CCH, _H), jnp.float32),
            pltpu.VMEM((_CCH, _H), jnp.float32),
            pltpu.SemaphoreType.DMA,
            pltpu.SemaphoreType.DMA,
        ],
    )(_combine_body)


def _combine_body(dstT_hbm, ys_hbm, out_hbm, idxA_v, idxB_v, bufA, bufB,
                  semA, semB):
    wid = lax.axis_index("s") * 2 + lax.axis_index("c")
    tbase = wid * _TPT
    pltpu.sync_copy(dstT_hbm.at[0, pl.ds(tbase, _TPT)], idxA_v)
    pltpu.sync_copy(dstT_hbm.at[1, pl.ds(tbase, _TPT)], idxB_v)

    for c2 in range(_TPT // _CCH):
        cpA = pltpu.async_copy(ys_hbm.at[idxA_v.at[pl.ds(c2 * _CCH, _CCH)]],
                               bufA, semA)
        cpB = pltpu.async_copy(ys_hbm.at[idxB_v.at[pl.ds(c2 * _CCH, _CCH)]],
                               bufB, semB)
        cpA.wait()
        cpB.wait()

        def add_body(j, _):
            r = j >> 6
            c = j & 63
            plsc.addupdate(bufA.at[r, pl.ds(c * 16, 16)],
                           bufB[r, pl.ds(c * 16, 16)])
            return 0
        lax.fori_loop(0, _CCH * (_H // 16), add_body, 0)
        pltpu.sync_copy(bufA, out_hbm.at[pl.ds(tbase + c2 * _CCH, _CCH)])


@jax.jit
def kernel(hidden_states, topk_weights, topk_ids, w13, w2, gate_up_lora_a,
           gate_up_lora_b, down_lora_a, down_lora_b):
    dst_t, be, stok, sw2d = _run_meta(topk_ids.T, topk_weights)
    be24 = be.reshape(-1)
    xs = _make_dispatch()(stok.reshape(_P), hidden_states)
    ys = _run_gmm(be24, xs, sw2d, w13, w2, gate_up_lora_a, gate_up_lora_b,
                  down_lora_a, down_lora_b)
    return _make_combine()(dst_t, ys)


# speedup vs baseline: 2.1510x; 1.5550x over previous
import functools

import jax
import jax.numpy as jnp
from jax import lax
from jax.experimental import pallas as pl
from jax.experimental.pallas import tpu as pltpu
from jax.experimental.pallas import tpu_sc as plsc

_E = 8
_K = 2
_H = 1024
_I = 1024
_R = 16
_T = 2048
_BT = 256
_NB = (_T * _K) // _BT + _E
_P = _NB * _BT
_NW = 32
_SPT = _P // _NW
_TPT = _T // _NW
_GCH = 48
_CCH = 32

_MESH = dict(core_axis_name="c", subcore_axis_name="s", num_cores=2,
             num_subcores=16)


def _meta_body(idst_ref, tw_ref, dstt_ref, be_ref, stok_ref, sw_ref):
    f32 = jnp.float32
    i32 = jnp.int32
    dn = (((1,), (0,)), ((), ()))

    iota_e_col = lax.broadcasted_iota(i32, (_E, 1), 0)
    triu = (lax.broadcasted_iota(i32, (_BT, _BT), 0) <
            lax.broadcasted_iota(i32, (_BT, _BT), 1)).astype(f32)
    ohs, excls, tots = [], [], []
    for r in range(_K):
        oh = (idst_ref[r:r + 1, :] == iota_e_col).astype(f32)
        carry = jnp.zeros((_E, 1), f32)
        tiles = []
        for i in range(_T // _BT):
            oht = oh[:, i * _BT:(i + 1) * _BT]
            tiles.append(lax.dot_general(oht, triu, dn,
                                         preferred_element_type=f32) + carry)
            carry = carry + jnp.sum(oht, axis=1, keepdims=True)
        ohs.append(oh)
        excls.append(jnp.concatenate(tiles, axis=1))
        tots.append(carry)

    counts = tots[0] + tots[1]
    nblk = jnp.floor((counts + float(_BT - 1)) / float(_BT))
    ltt = (lax.broadcasted_iota(i32, (_E, _E), 1) <
           lax.broadcasted_iota(i32, (_E, _E), 0)).astype(f32)
    blk_off = lax.dot_general(ltt, nblk, dn, preferred_element_type=f32)
    pad_off = blk_off * float(_BT)

    dst0 = jnp.sum(ohs[0] * (pad_off + excls[0]), axis=0, keepdims=True)
    dst1 = jnp.sum(ohs[1] * (pad_off + tots[0] + excls[1]), axis=0,
                   keepdims=True)
    dstt_ref[...] = jnp.concatenate([dst0, dst1], axis=0).astype(i32)

    tw = tw_ref[...]
    tww = tw / jnp.sum(tw, axis=1, keepdims=True)
    pw2 = jnp.concatenate([tww[:, 0:1], tww[:, 1:2]], axis=0)
    whi = pw2.astype(jnp.bfloat16).astype(f32)
    wlo = pw2 - whi
    tok = lax.broadcasted_iota(i32, (_T * _K, 1), 0) & (_T - 1)
    thi = (tok >> 7).astype(f32)
    tlo = (tok & 127).astype(f32)
    tv = jnp.concatenate([thi, tlo, whi, wlo], axis=1)

    dst_all = jnp.concatenate([dst0, dst1], axis=1)
    slot_col = lax.broadcasted_iota(i32, (_BT, 1), 0).astype(f32)
    for b in range(_NB):
        ind = (dst_all == (float(b * _BT) + slot_col)).astype(f32)
        res = lax.dot_general(ind, tv, dn, preferred_element_type=f32)
        stok_ref[pl.ds(b * _BT, _BT), :] = (
            res[:, 0:1] * 128.0 + res[:, 1:2]).astype(i32)
        sw_ref[pl.ds(b * _BT, _BT), :] = res[:, 2:3] + res[:, 3:4]

    bi = lax.broadcasted_iota(i32, (_E, _NB), 1).astype(f32)
    ge = (bi >= jnp.broadcast_to(blk_off, (_E, _NB))).astype(f32)
    be_val = jnp.sum(ge, axis=0, keepdims=True) - 1.0
    total_blk = jnp.sum(nblk, axis=0, keepdims=True)
    active = bi[0:1, :] < jnp.broadcast_to(total_blk, (1, _NB))
    be_ref[...] = jnp.where(active, be_val, -1.0).astype(i32)


def _run_meta(topk_ids_t, topk_weights):
    return pl.pallas_call(
        _meta_body,
        out_shape=(
            jax.ShapeDtypeStruct((_K, _T), jnp.int32),
            jax.ShapeDtypeStruct((1, _NB), jnp.int32),
            jax.ShapeDtypeStruct((_P, 1), jnp.int32),
            jax.ShapeDtypeStruct((_P, 1), jnp.float32),
        ),
    )(topk_ids_t, topk_weights)


def _gmm_body(be_ref, stok_ref, sw_ref, hid_ref, w13_ref, w2_ref, gua_ref,
              gub_ref, da_ref, db_ref, out_ref, hidb_ref):
    i = pl.program_id(0)
    active = be_ref[i] >= 0
    bf = jnp.bfloat16

    @pl.when(i == 0)
    def _():
        hidb_ref[...] = hid_ref[...].astype(bf)

    @pl.when(active)
    def _():
        tok_row = lax.broadcasted_iota(jnp.int32, (1, _T), 1)
        ind = (stok_ref[...] == tok_row).astype(bf)
        dg = (((1,), (0,)), ((), ()))
        x = lax.dot_general(ind, hidb_ref[...], dg,
                            preferred_element_type=jnp.float32
                            ).astype(bf)
        dn = (((1,), (1,)), ((), ()))
        gate_up = lax.dot_general(x, w13_ref[0].astype(bf), dn,
                                  preferred_element_type=jnp.float32)
        mid = lax.dot_general(x, gua_ref[0].astype(bf), dn,
                              preferred_element_type=jnp.float32)
        gate_up = gate_up + lax.dot_general(mid.astype(bf),
                                            gub_ref[0].astype(bf), dn,
                                            preferred_element_type=jnp.float32)
        gate = gate_up[:, :_I]
        up = gate_up[:, _I:]
        act = (gate * jax.nn.sigmoid(gate) * up).astype(bf)
        down = lax.dot_general(act, w2_ref[0].astype(bf), dn,
                               preferred_element_type=jnp.float32)
        dmid = lax.dot_general(act, da_ref[0].astype(bf), dn,
                               preferred_element_type=jnp.float32)
        down = down + lax.dot_general(dmid.astype(bf), db_ref[0].astype(bf),
                                      dn, preferred_element_type=jnp.float32)
        out_ref[...] = sw_ref[...] * down

    @pl.when(jnp.logical_not(active))
    def _():
        out_ref[...] = jnp.zeros_like(out_ref)


def _run_gmm(be, stok, sw2d, hidden, w13, w2, gua, gub, da, db):
    def emap(i, be_r):
        return (jnp.where(be_r[i] < 0, _E - 1, be_r[i]), 0, 0)

    grid_spec = pltpu.PrefetchScalarGridSpec(
        num_scalar_prefetch=1,
        grid=(_NB,),
        in_specs=[
            pl.BlockSpec((_BT, 1), lambda i, be_r: (i, 0)),
            pl.BlockSpec((_BT, 1), lambda i, be_r: (i, 0)),
            pl.BlockSpec((_T, _H), lambda i, be_r: (0, 0)),
            pl.BlockSpec((1, 2 * _I, _H), emap),
            pl.BlockSpec((1, _H, _I), emap),
            pl.BlockSpec((1, _R, _H), emap),
            pl.BlockSpec((1, 2 * _I, _R), emap),
            pl.BlockSpec((1, _R, _I), emap),
            pl.BlockSpec((1, _H, _R), emap),
        ],
        out_specs=pl.BlockSpec((_BT, _H), lambda i, be_r: (i, 0)),
        scratch_shapes=[pltpu.VMEM((_T, _H), jnp.bfloat16)],
    )
    return pl.pallas_call(
        _gmm_body,
        grid_spec=grid_spec,
        out_shape=jax.ShapeDtypeStruct((_P, _H), jnp.float32),
    )(be, stok, sw2d, hidden, w13, w2, gua, gub, da, db)


@functools.lru_cache(maxsize=None)
def _make_combine():
    return functools.partial(
        pl.kernel,
        out_type=jax.ShapeDtypeStruct((_T, _H), jnp.float32),
        mesh=plsc.VectorSubcoreMesh(**_MESH),
        compiler_params=pltpu.CompilerParams(needs_layout_passes=False),
        scratch_types=[
            pltpu.VMEM((_TPT,), jnp.int32),
            pltpu.VMEM((_TPT,), jnp.int32),
            pltpu.VMEM((_CCH, _H), jnp.float32),
            pltpu.VMEM((_CCH, _H), jnp.float32),
            pltpu.SemaphoreType.DMA,
            pltpu.SemaphoreType.DMA,
        ],
    )(_combine_body)


def _combine_body(dstT_hbm, ys_hbm, out_hbm, idxA_v, idxB_v, bufA, bufB,
                  semA, semB):
    wid = lax.axis_index("s") * 2 + lax.axis_index("c")
    tbase = wid * _TPT
    pltpu.sync_copy(dstT_hbm.at[0, pl.ds(tbase, _TPT)], idxA_v)
    pltpu.sync_copy(dstT_hbm.at[1, pl.ds(tbase, _TPT)], idxB_v)

    for c2 in range(_TPT // _CCH):
        cpA = pltpu.async_copy(ys_hbm.at[idxA_v.at[pl.ds(c2 * _CCH, _CCH)]],
                               bufA, semA)
        cpB = pltpu.async_copy(ys_hbm.at[idxB_v.at[pl.ds(c2 * _CCH, _CCH)]],
                               bufB, semB)
        cpA.wait()
        cpB.wait()

        def add_body(j, _):
            r = j >> 6
            c = j & 63
            plsc.addupdate(bufA.at[r, pl.ds(c * 16, 16)],
                           bufB[r, pl.ds(c * 16, 16)])
            return 0
        lax.fori_loop(0, _CCH * (_H // 16), add_body, 0)
        pltpu.sync_copy(bufA, out_hbm.at[pl.ds(tbase + c2 * _CCH, _CCH)])


@jax.jit
def kernel(hidden_states, topk_weights, topk_ids, w13, w2, gate_up_lora_a,
           gate_up_lora_b, down_lora_a, down_lora_b):
    dst_t, be, stok, sw2d = _run_meta(topk_ids.T, topk_weights)
    be24 = be.reshape(-1)
    ys = _run_gmm(be24, stok, sw2d, hidden_states, w13, w2, gate_up_lora_a,
                  gate_up_lora_b, down_lora_a, down_lora_b)
    return _make_combine()(dst_t, ys)
